# baseline (speedup 1.0000x reference)
import functools

import jax
import jax.numpy as jnp
from jax import lax
from jax.experimental import pallas as pl
from jax.experimental.pallas import tpu as pltpu
from jax.experimental.pallas import tpu_sc as plsc

B = 4096
D = 32
NC = 2
NS = 16
NW = NC * NS
BPW = B // NW
L = 16


def _gather_rows(idx_ref, table_hbm, rows, sem):
    copies = []
    for c in range(BPW // L):
        chunk = idx_ref[pl.ds(c * L, L)]
        for l in range(L):
            r = jnp.sum(jnp.where(lax.iota(jnp.int32, L) == l, chunk, 0))
            p = c * L + l
            copies.append(pltpu.async_copy(
                table_hbm.at[pl.ds(r * D, D)], rows.at[pl.ds(p * D, D)], sem))
    for cp in copies:
        cp.wait()


def _sc_gather_body(users_hbm, items_hbm, uf_hbm, if_hbm, u_out, v_out,
                    uidx, urows, iidx, irows, usem, isem):
    wid = lax.axis_index("s") * NC + lax.axis_index("c")
    base = wid * BPW
    pltpu.sync_copy(users_hbm.at[pl.ds(base, BPW)], uidx)
    pltpu.sync_copy(items_hbm.at[pl.ds(base, BPW)], iidx)
    _gather_rows(uidx, uf_hbm, urows, usem)
    _gather_rows(iidx, if_hbm, irows, isem)
    pltpu.sync_copy(urows, u_out.at[pl.ds(base * D, BPW * D)])
    pltpu.sync_copy(irows, v_out.at[pl.ds(base * D, BPW * D)])


_sc_gather = functools.partial(
    pl.kernel,
    mesh=plsc.VectorSubcoreMesh(core_axis_name="c", subcore_axis_name="s"),
    out_type=[
        jax.ShapeDtypeStruct((B * D,), jnp.float32),
        jax.ShapeDtypeStruct((B * D,), jnp.float32),
    ],
    scratch_types=[
        pltpu.VMEM((BPW,), jnp.int32),
        pltpu.VMEM((BPW * D,), jnp.float32),
        pltpu.VMEM((BPW,), jnp.int32),
        pltpu.VMEM((BPW * D,), jnp.float32),
        pltpu.SemaphoreType.DMA,
        pltpu.SemaphoreType.DMA,
    ],
    compiler_params=pltpu.CompilerParams(needs_layout_passes=False),
)(_sc_gather_body)


def _mm_body(u_ref, v_ref, o_ref):
    o_ref[...] = lax.dot_general(
        u_ref[...], v_ref[...],
        (((1,), (1,)), ((), ())),
        preferred_element_type=jnp.float32,
    )


BM = 512
BN = 1024


def _tc_matmul(u, v):
    return pl.pallas_call(
        _mm_body,
        grid=(B // BM, B // BN),
        in_specs=[
            pl.BlockSpec((BM, D), lambda i, j: (i, 0)),
            pl.BlockSpec((BN, D), lambda i, j: (j, 0)),
        ],
        out_specs=pl.BlockSpec((BM, BN), lambda i, j: (i, j)),
        out_shape=jax.ShapeDtypeStruct((B, B), jnp.float32),
    )(u, v)


def kernel(users, items, user_factors, item_factors):
    u, v = _sc_gather(users.astype(jnp.int32), items.astype(jnp.int32),
                      user_factors.reshape(-1), item_factors.reshape(-1))
    return _tc_matmul(u.reshape(B, D), v.reshape(B, D))


# speedup vs baseline: 1.4870x; 1.4870x over previous
import functools

import jax
import jax.numpy as jnp
from jax import lax
from jax.experimental import pallas as pl
from jax.experimental.pallas import tpu as pltpu
from jax.experimental.pallas import tpu_sc as plsc

B = 4096
D = 32
NC = 2
NS = 16
NW = NC * NS
BPW = B // NW
L = 16


def _gather_rows(idx_ref, table_hbm, rows, sem):
    copies = []
    for c in range(BPW // L):
        chunk = idx_ref[pl.ds(c * L, L)]
        for l in range(L):
            r = lax.squeeze(lax.slice(chunk, (l,), (l + 1,)), (0,))
            p = c * L + l
            copies.append(pltpu.async_copy(
                table_hbm.at[pl.ds(r, 1)], rows.at[pl.ds(p, 1)], sem))
    for cp in copies:
        cp.wait()


def _sc_gather_body(users_hbm, items_hbm, uf_hbm, if_hbm, u_out, v_out,
                    uidx, urows, iidx, irows, usem, isem):
    wid = lax.axis_index("s") * NC + lax.axis_index("c")
    base = wid * BPW
    pltpu.sync_copy(users_hbm.at[pl.ds(base, BPW)], uidx)
    pltpu.sync_copy(items_hbm.at[pl.ds(base, BPW)], iidx)
    _gather_rows(uidx, uf_hbm, urows, usem)
    _gather_rows(iidx, if_hbm, irows, isem)
    pltpu.sync_copy(urows, u_out.at[pl.ds(base, BPW)])
    pltpu.sync_copy(irows, v_out.at[pl.ds(base, BPW)])


_sc_gather = functools.partial(
    pl.kernel,
    mesh=plsc.VectorSubcoreMesh(core_axis_name="c", subcore_axis_name="s"),
    out_type=[
        jax.ShapeDtypeStruct((B, D), jnp.float32),
        jax.ShapeDtypeStruct((B, D), jnp.float32),
    ],
    scratch_types=[
        pltpu.VMEM((BPW,), jnp.int32),
        pltpu.VMEM((BPW, D), jnp.float32),
        pltpu.VMEM((BPW,), jnp.int32),
        pltpu.VMEM((BPW, D), jnp.float32),
        pltpu.SemaphoreType.DMA,
        pltpu.SemaphoreType.DMA,
    ],
)(_sc_gather_body)


def _mm_body(u_ref, v_ref, o_ref):
    o_ref[...] = lax.dot_general(
        u_ref[...], v_ref[...],
        (((1,), (1,)), ((), ())),
        preferred_element_type=jnp.float32,
    )


BM = 512
BN = 1024


def _tc_matmul(u, v):
    return pl.pallas_call(
        _mm_body,
        grid=(B // BM, B // BN),
        in_specs=[
            pl.BlockSpec((BM, D), lambda i, j: (i, 0)),
            pl.BlockSpec((BN, D), lambda i, j: (j, 0)),
        ],
        out_specs=pl.BlockSpec((BM, BN), lambda i, j: (i, j)),
        out_shape=jax.ShapeDtypeStruct((B, B), jnp.float32),
    )(u, v)


def kernel(users, items, user_factors, item_factors):
    u, v = _sc_gather(users.astype(jnp.int32), items.astype(jnp.int32),
                      user_factors, item_factors)
    return _tc_matmul(u, v)


# speedup vs baseline: 4.8071x; 3.2327x over previous
import functools

import jax
import jax.numpy as jnp
from jax import lax
from jax.experimental import pallas as pl
from jax.experimental.pallas import tpu as pltpu
from jax.experimental.pallas import tpu_sc as plsc

B = 4096
D = 32
FG = 8
NFG = D // FG
W = 128
NC = 2
NS = 16
NW = NC * NS
BPW = B // NW
L = 16
NCH = BPW // L
DE = D * L


def _gather_table(idx_ref, table_hbm, buf, stage, sem):
    def chunk_body(t, carry):
        a = t // NCH
        c = lax.rem(t, jnp.int32(NCH))
        chunk = idx_ref[pl.ds(c * L, L)]
        copies = []
        for k in range(L):
            r = lax.squeeze(lax.slice(chunk, (k,), (k + 1,)), (0,))
            off = pl.multiple_of((r // W) * W, W)
            copies.append(pltpu.async_copy(
                table_hbm.at[pl.ds(a * FG, FG), pl.ds(off, W)],
                buf.at[pl.ds(k * FG, FG)], sem))
        for cp in copies:
            cp.wait()
        for k in range(L):
            r = lax.squeeze(lax.slice(chunk, (k,), (k + 1,)), (0,))
            l = lax.rem(r, jnp.int32(W))
            p = c * L + k
            for f in range(FG):
                v = buf[k * FG + f, pl.ds(l, L)]
                stage[p, pl.ds((a * FG + f) * L, L)] = v
        return carry

    lax.fori_loop(0, NFG * NCH, chunk_body, 0)


def _sc_gather_body(users_hbm, items_hbm, uf_hbm, if_hbm, u_out, v_out,
                    uidx, iidx, buf, stage, sem):
    wid = lax.axis_index("s") * NC + lax.axis_index("c")
    base = wid * BPW
    pltpu.sync_copy(users_hbm.at[pl.ds(base, BPW)], uidx)
    pltpu.sync_copy(items_hbm.at[pl.ds(base, BPW)], iidx)
    _gather_table(uidx, uf_hbm, buf, stage, sem)
    pltpu.sync_copy(stage, u_out.at[pl.ds(base, BPW)])
    _gather_table(iidx, if_hbm, buf, stage, sem)
    pltpu.sync_copy(stage, v_out.at[pl.ds(base, BPW)])


_sc_gather = functools.partial(
    pl.kernel,
    mesh=plsc.VectorSubcoreMesh(core_axis_name="c", subcore_axis_name="s"),
    out_type=[
        jax.ShapeDtypeStruct((B, DE), jnp.float32),
        jax.ShapeDtypeStruct((B, DE), jnp.float32),
    ],
    scratch_types=[
        pltpu.VMEM((BPW,), jnp.int32),
        pltpu.VMEM((BPW,), jnp.int32),
        pltpu.VMEM((L * FG + 1, W), jnp.float32),
        pltpu.VMEM((BPW, DE), jnp.float32),
        pltpu.SemaphoreType.DMA,
    ],
)(_sc_gather_body)


def _compact_body(ue_ref, ve_ref, u_ref, v_ref):
    sel = (lax.broadcasted_iota(jnp.int32, (DE, D), 0)
           == lax.broadcasted_iota(jnp.int32, (DE, D), 1) * L
           ).astype(jnp.float32)
    u_ref[...] = jnp.dot(ue_ref[...], sel, preferred_element_type=jnp.float32)
    v_ref[...] = jnp.dot(ve_ref[...], sel, preferred_element_type=jnp.float32)


def _tc_compact(ue, ve):
    return pl.pallas_call(
        _compact_body,
        out_shape=[jax.ShapeDtypeStruct((B, D), jnp.float32),
                   jax.ShapeDtypeStruct((B, D), jnp.float32)],
    )(ue, ve)


def _mm_body(u_ref, v_ref, o_ref):
    o_ref[...] = lax.dot_general(
        u_ref[...], v_ref[...],
        (((1,), (1,)), ((), ())),
        preferred_element_type=jnp.float32,
    )


BM = 512
BN = 1024


def _tc_matmul(u, v):
    return pl.pallas_call(
        _mm_body,
        grid=(B // BM, B // BN),
        in_specs=[
            pl.BlockSpec((BM, D), lambda i, j: (i, 0)),
            pl.BlockSpec((BN, D), lambda i, j: (j, 0)),
        ],
        out_specs=pl.BlockSpec((BM, BN), lambda i, j: (i, j)),
        out_shape=jax.ShapeDtypeStruct((B, B), jnp.float32),
    )(u, v)


def kernel(users, items, user_factors, item_factors):
    ue, ve = _sc_gather(users.astype(jnp.int32), items.astype(jnp.int32),
                        user_factors.T, item_factors.T)
    u, v = _tc_compact(ue, ve)
    return _tc_matmul(u, v)


# speedup vs baseline: 5.1326x; 1.0677x over previous
import functools

import jax
import jax.numpy as jnp
from jax import lax
from jax.experimental import pallas as pl
from jax.experimental.pallas import tpu as pltpu
from jax.experimental.pallas import tpu_sc as plsc

B = 4096
D = 32
FG = 8
NFG = D // FG
W = 128
NC = 2
NS = 16
NW = NC * NS
BPW = B // NW
L = 16
NCH = BPW // L
DE = D * L


def _gather_table(idx_ref, table_hbm, buf, stage, sem):
    def chunk_body(t, carry):
        a = t // NCH
        c = lax.rem(t, jnp.int32(NCH))
        chunk = idx_ref[pl.ds(c * L, L)]
        copies = []
        for k in range(L):
            r = lax.squeeze(lax.slice(chunk, (k,), (k + 1,)), (0,))
            off = pl.multiple_of((r // W) * W, W)
            copies.append(pltpu.async_copy(
                table_hbm.at[pl.ds(a * FG, FG), pl.ds(off, W)],
                buf.at[pl.ds(k * FG, FG)], sem))
        for cp in copies:
            cp.wait()
        for k in range(L):
            r = lax.squeeze(lax.slice(chunk, (k,), (k + 1,)), (0,))
            l = lax.rem(r, jnp.int32(W))
            p = c * L + k
            for f in range(FG):
                v = buf[k * FG + f, pl.ds(l, L)]
                stage[p, pl.ds((a * FG + f) * L, L)] = v
        return carry

    lax.fori_loop(0, NFG * NCH, chunk_body, 0)


def _sc_gather_body(users_hbm, items_hbm, uf_hbm, if_hbm, u_out, v_out,
                    uidx, iidx, buf, stage, sem):
    wid = lax.axis_index("s") * NC + lax.axis_index("c")
    base = wid * BPW
    pltpu.sync_copy(users_hbm.at[pl.ds(base, BPW)], uidx)
    pltpu.sync_copy(items_hbm.at[pl.ds(base, BPW)], iidx)
    _gather_table(uidx, uf_hbm, buf, stage, sem)
    pltpu.sync_copy(stage, u_out.at[pl.ds(base, BPW)])
    _gather_table(iidx, if_hbm, buf, stage, sem)
    pltpu.sync_copy(stage, v_out.at[pl.ds(base, BPW)])


_sc_gather = functools.partial(
    pl.kernel,
    mesh=plsc.VectorSubcoreMesh(core_axis_name="c", subcore_axis_name="s"),
    out_type=[
        jax.ShapeDtypeStruct((B, DE), jnp.float32),
        jax.ShapeDtypeStruct((B, DE), jnp.float32),
    ],
    scratch_types=[
        pltpu.VMEM((BPW,), jnp.int32),
        pltpu.VMEM((BPW,), jnp.int32),
        pltpu.VMEM((L * FG + 1, W), jnp.float32),
        pltpu.VMEM((BPW, DE), jnp.float32),
        pltpu.SemaphoreType.DMA,
    ],
)(_sc_gather_body)


BM = 512
BN = 1024


def _mm_body(ue_ref, ve_ref, o_ref, u_s, v_s):
    i = pl.program_id(0)
    j = pl.program_id(1)

    @pl.when((i == 0) & (j == 0))
    def _():
        sel = (lax.broadcasted_iota(jnp.int32, (DE, D), 0)
               == lax.broadcasted_iota(jnp.int32, (DE, D), 1) * L
               ).astype(jnp.float32)
        u_s[...] = jnp.dot(ue_ref[...], sel,
                           preferred_element_type=jnp.float32)
        v_s[...] = jnp.dot(ve_ref[...], sel,
                           preferred_element_type=jnp.float32)

    o_ref[...] = lax.dot_general(
        u_s[pl.ds(i * BM, BM), :], v_s[pl.ds(j * BN, BN), :],
        (((1,), (1,)), ((), ())),
        preferred_element_type=jnp.float32,
    )


def _tc_matmul(ue, ve):
    return pl.pallas_call(
        _mm_body,
        grid=(B // BM, B // BN),
        in_specs=[
            pl.BlockSpec((B, DE), lambda i, j: (0, 0)),
            pl.BlockSpec((B, DE), lambda i, j: (0, 0)),
        ],
        out_specs=pl.BlockSpec((BM, BN), lambda i, j: (i, j)),
        out_shape=jax.ShapeDtypeStruct((B, B), jnp.float32),
        scratch_shapes=[
            pltpu.VMEM((B, D), jnp.float32),
            pltpu.VMEM((B, D), jnp.float32),
        ],
    )(ue, ve)


def kernel(users, items, user_factors, item_factors):
    ue, ve = _sc_gather(users.astype(jnp.int32), items.astype(jnp.int32),
                        user_factors.T, item_factors.T)
    return _tc_matmul(ue, ve)


# speedup vs baseline: 6.9454x; 1.3532x over previous
import functools

import jax
import jax.numpy as jnp
from jax import lax
from jax.experimental import pallas as pl
from jax.experimental.pallas import tpu as pltpu
from jax.experimental.pallas import tpu_sc as plsc

B = 4096
D = 32
FG = 8
NFG = D // FG
W = 128
NC = 2
NS = 16
NW = NC * NS
BPW = B // NW
L = 16
NCH = BPW // L
DE = D * L


NSTEP = NFG * NCH


def _fire(idx_ref, table_hbm, s, buf, sem):
    a = s // NCH
    c = lax.rem(s, jnp.int32(NCH))
    chunk = idx_ref[pl.ds(c * L, L)]
    for k in range(L):
        r = lax.squeeze(lax.slice(chunk, (k,), (k + 1,)), (0,))
        off = pl.multiple_of((r // W) * W, W)
        pltpu.async_copy(
            table_hbm.at[pl.ds(a * FG, FG), pl.ds(off, W)],
            buf.at[pl.ds(k * FG, FG)], sem)


def _drain_extract(idx_ref, table_hbm, s, buf, sem, stage):
    for k in range(L):
        pltpu.make_async_copy(
            table_hbm.at[pl.ds(0, FG), pl.ds(0, W)],
            buf.at[pl.ds(k * FG, FG)], sem).wait()
    a = s // NCH
    c = lax.rem(s, jnp.int32(NCH))
    chunk = idx_ref[pl.ds(c * L, L)]
    for k in range(L):
        r = lax.squeeze(lax.slice(chunk, (k,), (k + 1,)), (0,))
        l = lax.rem(r, jnp.int32(W))
        p = c * L + k
        for f in range(FG):
            v = buf[k * FG + f, pl.ds(l, L)]
            stage[p, pl.ds((a * FG + f) * L, L)] = v


def _gather_table(idx_ref, table_hbm, bufa, bufb, stage, sema, semb):
    _fire(idx_ref, table_hbm, jnp.int32(0), bufa, sema)
    _fire(idx_ref, table_hbm, jnp.int32(1), bufb, semb)

    def pair_body(t, carry):
        s = t * 2
        _drain_extract(idx_ref, table_hbm, s, bufa, sema, stage)

        @pl.when(s + 2 < NSTEP)
        def _():
            _fire(idx_ref, table_hbm, s + 2, bufa, sema)

        _drain_extract(idx_ref, table_hbm, s + 1, bufb, semb, stage)

        @pl.when(s + 3 < NSTEP)
        def _():
            _fire(idx_ref, table_hbm, s + 3, bufb, semb)

        return carry

    lax.fori_loop(0, NSTEP // 2, pair_body, 0)


def _sc_gather_body(users_hbm, items_hbm, uf_hbm, if_hbm, u_out, v_out,
                    uidx, iidx, bufa, bufb, stage, sema, semb):
    wid = lax.axis_index("s") * NC + lax.axis_index("c")
    base = wid * BPW
    pltpu.sync_copy(users_hbm.at[pl.ds(base, BPW)], uidx)
    pltpu.sync_copy(items_hbm.at[pl.ds(base, BPW)], iidx)
    _gather_table(uidx, uf_hbm, bufa, bufb, stage, sema, semb)
    pltpu.sync_copy(stage, u_out.at[pl.ds(base, BPW)])
    _gather_table(iidx, if_hbm, bufa, bufb, stage, sema, semb)
    pltpu.sync_copy(stage, v_out.at[pl.ds(base, BPW)])


_sc_gather = functools.partial(
    pl.kernel,
    mesh=plsc.VectorSubcoreMesh(core_axis_name="c", subcore_axis_name="s"),
    out_type=[
        jax.ShapeDtypeStruct((B, DE), jnp.float32),
        jax.ShapeDtypeStruct((B, DE), jnp.float32),
    ],
    scratch_types=[
        pltpu.VMEM((BPW,), jnp.int32),
        pltpu.VMEM((BPW,), jnp.int32),
        pltpu.VMEM((L * FG + 1, W), jnp.float32),
        pltpu.VMEM((L * FG + 1, W), jnp.float32),
        pltpu.VMEM((BPW, DE), jnp.float32),
        pltpu.SemaphoreType.DMA,
        pltpu.SemaphoreType.DMA,
    ],
)(_sc_gather_body)


BM = 512
BN = 1024


def _mm_body(ue_ref, ve_ref, o_ref, u_s, v_s):
    i = pl.program_id(0)
    j = pl.program_id(1)

    @pl.when((i == 0) & (j == 0))
    def _():
        sel = (lax.broadcasted_iota(jnp.int32, (DE, D), 0)
               == lax.broadcasted_iota(jnp.int32, (DE, D), 1) * L
               ).astype(jnp.float32)
        u_s[...] = jnp.dot(ue_ref[...], sel,
                           preferred_element_type=jnp.float32)
        v_s[...] = jnp.dot(ve_ref[...], sel,
                           preferred_element_type=jnp.float32)

    o_ref[...] = lax.dot_general(
        u_s[pl.ds(i * BM, BM), :], v_s[pl.ds(j * BN, BN), :],
        (((1,), (1,)), ((), ())),
        preferred_element_type=jnp.float32,
    )


def _tc_matmul(ue, ve):
    return pl.pallas_call(
        _mm_body,
        grid=(B // BM, B // BN),
        in_specs=[
            pl.BlockSpec((B, DE), lambda i, j: (0, 0)),
            pl.BlockSpec((B, DE), lambda i, j: (0, 0)),
        ],
        out_specs=pl.BlockSpec((BM, BN), lambda i, j: (i, j)),
        out_shape=jax.ShapeDtypeStruct((B, B), jnp.float32),
        scratch_shapes=[
            pltpu.VMEM((B, D), jnp.float32),
            pltpu.VMEM((B, D), jnp.float32),
        ],
    )(ue, ve)


def kernel(users, items, user_factors, item_factors):
    ue, ve = _sc_gather(users.astype(jnp.int32), items.astype(jnp.int32),
                        user_factors.T, item_factors.T)
    return _tc_matmul(ue, ve)


# speedup vs baseline: 7.3697x; 1.0611x over previous
import functools

import jax
import jax.numpy as jnp
from jax import lax
from jax.experimental import pallas as pl
from jax.experimental.pallas import tpu as pltpu
from jax.experimental.pallas import tpu_sc as plsc

B = 4096
D = 32
FG = 8
NFG = D // FG
W = 128
NC = 2
NS = 16
NW = NC * NS
BPW = B // NW
L = 16
NCH = BPW // L
DE = D * L


NSTEP = NFG * NCH


def _fire(idx_ref, table_hbm, s, buf, sem):
    a = s // NCH
    c = lax.rem(s, jnp.int32(NCH))
    chunk = idx_ref[pl.ds(c * L, L)]
    for k in range(L):
        r = lax.squeeze(lax.slice(chunk, (k,), (k + 1,)), (0,))
        off = pl.multiple_of((r // W) * W, W)
        pltpu.async_copy(
            table_hbm.at[pl.ds(a * FG, FG), pl.ds(off, W)],
            buf.at[pl.ds(k * FG, FG)], sem)


def _drain_extract(idx_ref, table_hbm, s, buf, sem, stage):
    for k in range(L):
        pltpu.make_async_copy(
            table_hbm.at[pl.ds(0, FG), pl.ds(0, W)],
            buf.at[pl.ds(k * FG, FG)], sem).wait()
    a = s // NCH
    c = lax.rem(s, jnp.int32(NCH))
    chunk = idx_ref[pl.ds(c * L, L)]
    for k in range(L):
        r = lax.squeeze(lax.slice(chunk, (k,), (k + 1,)), (0,))
        l = lax.rem(r, jnp.int32(W))
        p = c * L + k
        for f in range(FG):
            v = buf[k * FG + f, pl.ds(l, L)]
            stage[p, pl.ds((a * FG + f) * L, L)] = v


def _gather_table(idx_ref, table_hbm, bufa, bufb, stage, sema, semb):
    _fire(idx_ref, table_hbm, jnp.int32(0), bufa, sema)
    _fire(idx_ref, table_hbm, jnp.int32(1), bufb, semb)

    def pair_body(t, carry):
        s = t * 2
        _drain_extract(idx_ref, table_hbm, s, bufa, sema, stage)

        @pl.when(s + 2 < NSTEP)
        def _():
            _fire(idx_ref, table_hbm, s + 2, bufa, sema)

        _drain_extract(idx_ref, table_hbm, s + 1, bufb, semb, stage)

        @pl.when(s + 3 < NSTEP)
        def _():
            _fire(idx_ref, table_hbm, s + 3, bufb, semb)

        return carry

    lax.fori_loop(0, NSTEP // 2, pair_body, 0)


def _sc_gather_body(users_hbm, items_hbm, uf_hbm, if_hbm, u_out, v_out,
                    uidx, iidx, bufa, bufb, stage, sema, semb):
    wid = lax.axis_index("s") * NC + lax.axis_index("c")
    base = wid * BPW
    pltpu.sync_copy(users_hbm.at[pl.ds(base, BPW)], uidx)
    pltpu.sync_copy(items_hbm.at[pl.ds(base, BPW)], iidx)
    _gather_table(uidx, uf_hbm, bufa, bufb, stage, sema, semb)
    pltpu.sync_copy(stage, u_out.at[pl.ds(base, BPW)])
    _gather_table(iidx, if_hbm, bufa, bufb, stage, sema, semb)
    pltpu.sync_copy(stage, v_out.at[pl.ds(base, BPW)])


_sc_gather = functools.partial(
    pl.kernel,
    mesh=plsc.VectorSubcoreMesh(core_axis_name="c", subcore_axis_name="s"),
    out_type=[
        jax.ShapeDtypeStruct((B, DE), jnp.float32),
        jax.ShapeDtypeStruct((B, DE), jnp.float32),
    ],
    scratch_types=[
        pltpu.VMEM((BPW,), jnp.int32),
        pltpu.VMEM((BPW,), jnp.int32),
        pltpu.VMEM((L * FG + 1, W), jnp.float32),
        pltpu.VMEM((L * FG + 1, W), jnp.float32),
        pltpu.VMEM((BPW, DE), jnp.float32),
        pltpu.SemaphoreType.DMA,
        pltpu.SemaphoreType.DMA,
    ],
)(_sc_gather_body)


BM = 1024
BN = 2048


def _mm_body(ue_ref, ve_ref, o_ref, u_s, v_s):
    i = pl.program_id(0)
    j = pl.program_id(1)

    @pl.when((i == 0) & (j == 0))
    def _():
        sel = (lax.broadcasted_iota(jnp.int32, (DE, D), 0)
               == lax.broadcasted_iota(jnp.int32, (DE, D), 1) * L
               ).astype(jnp.float32)
        u_s[...] = jnp.dot(ue_ref[...], sel,
                           preferred_element_type=jnp.float32)
        v_s[...] = jnp.dot(ve_ref[...], sel,
                           preferred_element_type=jnp.float32)

    o_ref[...] = lax.dot_general(
        u_s[pl.ds(i * BM, BM), :], v_s[pl.ds(j * BN, BN), :],
        (((1,), (1,)), ((), ())),
        preferred_element_type=jnp.float32,
    )


def _tc_matmul(ue, ve):
    return pl.pallas_call(
        _mm_body,
        grid=(B // BM, B // BN),
        in_specs=[
            pl.BlockSpec((B, DE), lambda i, j: (0, 0)),
            pl.BlockSpec((B, DE), lambda i, j: (0, 0)),
        ],
        out_specs=pl.BlockSpec((BM, BN), lambda i, j: (i, j)),
        out_shape=jax.ShapeDtypeStruct((B, B), jnp.float32),
        scratch_shapes=[
            pltpu.VMEM((B, D), jnp.float32),
            pltpu.VMEM((B, D), jnp.float32),
        ],
    )(ue, ve)


def kernel(users, items, user_factors, item_factors):
    ue, ve = _sc_gather(users.astype(jnp.int32), items.astype(jnp.int32),
                        user_factors.T, item_factors.T)
    return _tc_matmul(ue, ve)


# speedup vs baseline: 7.8132x; 1.0602x over previous
import functools

import jax
import jax.numpy as jnp
from jax import lax
from jax.experimental import pallas as pl
from jax.experimental.pallas import tpu as pltpu
from jax.experimental.pallas import tpu_sc as plsc

B = 4096
D = 32
FG = 8
NFG = D // FG
W = 128
NC = 2
NS = 16
NW = NC * NS
BPW = B // NW
L = 16
NCH = BPW // L
DE = 128


NSTEP = NFG * NCH


def _fire(idx_ref, table_hbm, s, buf, sem):
    a = s // NCH
    c = lax.rem(s, jnp.int32(NCH))
    chunk = idx_ref[pl.ds(c * L, L)]
    for k in range(L):
        r = lax.squeeze(lax.slice(chunk, (k,), (k + 1,)), (0,))
        off = pl.multiple_of((r // W) * W, W)
        pltpu.async_copy(
            table_hbm.at[pl.ds(a * FG, FG), pl.ds(off, W)],
            buf.at[pl.ds(k * FG, FG)], sem)


def _drain_extract(idx_ref, table_hbm, s, buf, sem, stage):
    for k in range(L):
        pltpu.make_async_copy(
            table_hbm.at[pl.ds(0, FG), pl.ds(0, W)],
            buf.at[pl.ds(k * FG, FG)], sem).wait()
    a = s // NCH
    c = lax.rem(s, jnp.int32(NCH))
    chunk = idx_ref[pl.ds(c * L, L)]
    for k in range(L):
        r = lax.squeeze(lax.slice(chunk, (k,), (k + 1,)), (0,))
        l = lax.rem(r, jnp.int32(W))
        p = c * L + k
        for f in range(FG):
            v = buf[k * FG + f, pl.ds(l, L)]
            stage[p, pl.ds(a * FG + f, L)] = v


def _gather_table(idx_ref, table_hbm, bufa, bufb, stage, sema, semb):
    _fire(idx_ref, table_hbm, jnp.int32(0), bufa, sema)
    _fire(idx_ref, table_hbm, jnp.int32(1), bufb, semb)

    def pair_body(t, carry):
        s = t * 2
        _drain_extract(idx_ref, table_hbm, s, bufa, sema, stage)

        @pl.when(s + 2 < NSTEP)
        def _():
            _fire(idx_ref, table_hbm, s + 2, bufa, sema)

        _drain_extract(idx_ref, table_hbm, s + 1, bufb, semb, stage)

        @pl.when(s + 3 < NSTEP)
        def _():
            _fire(idx_ref, table_hbm, s + 3, bufb, semb)

        return carry

    lax.fori_loop(0, NSTEP // 2, pair_body, 0)


def _sc_gather_body(users_hbm, items_hbm, uf_hbm, if_hbm, u_out, v_out,
                    uidx, iidx, bufa, bufb, stage, sema, semb):
    wid = lax.axis_index("s") * NC + lax.axis_index("c")
    base = wid * BPW
    pltpu.sync_copy(users_hbm.at[pl.ds(base, BPW)], uidx)
    pltpu.sync_copy(items_hbm.at[pl.ds(base, BPW)], iidx)
    _gather_table(uidx, uf_hbm, bufa, bufb, stage, sema, semb)
    pltpu.sync_copy(stage, u_out.at[pl.ds(base, BPW)])
    _gather_table(iidx, if_hbm, bufa, bufb, stage, sema, semb)
    pltpu.sync_copy(stage, v_out.at[pl.ds(base, BPW)])


_sc_gather = functools.partial(
    pl.kernel,
    mesh=plsc.VectorSubcoreMesh(core_axis_name="c", subcore_axis_name="s"),
    out_type=[
        jax.ShapeDtypeStruct((B, DE), jnp.float32),
        jax.ShapeDtypeStruct((B, DE), jnp.float32),
    ],
    scratch_types=[
        pltpu.VMEM((BPW,), jnp.int32),
        pltpu.VMEM((BPW,), jnp.int32),
        pltpu.VMEM((L * FG + 1, W), jnp.float32),
        pltpu.VMEM((L * FG + 1, W), jnp.float32),
        pltpu.VMEM((BPW, DE), jnp.float32),
        pltpu.SemaphoreType.DMA,
        pltpu.SemaphoreType.DMA,
    ],
)(_sc_gather_body)


BM = 1024
BN = 2048


def _mm_body(ue_ref, ve_ref, o_ref):
    i = pl.program_id(0)
    j = pl.program_id(1)
    o_ref[...] = lax.dot_general(
        ue_ref[pl.ds(i * BM, BM), pl.ds(0, D)],
        ve_ref[pl.ds(j * BN, BN), pl.ds(0, D)],
        (((1,), (1,)), ((), ())),
        preferred_element_type=jnp.float32,
    )


def _tc_matmul(ue, ve):
    return pl.pallas_call(
        _mm_body,
        grid=(B // BM, B // BN),
        in_specs=[
            pl.BlockSpec((B, DE), lambda i, j: (0, 0)),
            pl.BlockSpec((B, DE), lambda i, j: (0, 0)),
        ],
        out_specs=pl.BlockSpec((BM, BN), lambda i, j: (i, j)),
        out_shape=jax.ShapeDtypeStruct((B, B), jnp.float32),
    )(ue, ve)


def kernel(users, items, user_factors, item_factors):
    ue, ve = _sc_gather(users.astype(jnp.int32), items.astype(jnp.int32),
                        user_factors.T, item_factors.T)
    return _tc_matmul(ue, ve)
